# Initial kernel scaffold; baseline (speedup 1.0000x reference)
import jax
import jax.numpy as jnp
from jax.experimental import pallas as pl


def kernel(x, w1, b1, w2, b2):
    raise NotImplementedError("write your pallas kernel here")


# speedup vs baseline: 1.5002x; 1.5002x over previous
import jax
import jax.numpy as jnp
from jax.experimental import pallas as pl
from jax.experimental.pallas import tpu as pltpu


def _round_up(n, m):
    return ((n + m - 1) // m) * m


def _mlp_fused_kernel(x_ref, w1_ref, b1_ref, w2_ref, b2_ref, o_ref):
    x = x_ref[...].astype(jnp.bfloat16)
    h = jax.lax.dot_general(
        x, w1_ref[...],
        dimension_numbers=(((1,), (1,)), ((), ())),
        preferred_element_type=jnp.float32,
    )
    h = jnp.maximum(h + b1_ref[...], 0.0).astype(jnp.bfloat16)
    out = jax.lax.dot_general(
        h, w2_ref[...],
        dimension_numbers=(((1,), (1,)), ((), ())),
        preferred_element_type=jnp.float32,
    )
    o_ref[...] = (out + b2_ref[...]).astype(o_ref.dtype)


def kernel(x, w1, b1, w2, b2, *, tile_b=512):
    B, In = x.shape
    H, _ = w1.shape
    C, _ = w2.shape
    dt = x.dtype

    TB = min(tile_b, _round_up(B, 8))
    B_p = _round_up(B, TB)
    if B_p != B:
        x = jnp.pad(x, ((0, B_p - B), (0, 0)))

    w1b = w1.astype(jnp.bfloat16)
    w2b = w2.astype(jnp.bfloat16)
    b1r = b1.reshape(1, H)
    b2r = b2.reshape(1, C)

    grid = (B_p // TB,)
    flops = 2 * B_p * (In * H + H * C)
    bytes_accessed = 4 * (B_p * In + B_p * C + H + C) + 2 * (In * H + H * C)

    out = pl.pallas_call(
        _mlp_fused_kernel,
        out_shape=jax.ShapeDtypeStruct((B_p, C), dt),
        grid_spec=pltpu.PrefetchScalarGridSpec(
            num_scalar_prefetch=0,
            grid=grid,
            in_specs=[
                pl.BlockSpec((TB, In), lambda i: (i, 0)),
                pl.BlockSpec((H, In), lambda i: (0, 0)),
                pl.BlockSpec((1, H), lambda i: (0, 0)),
                pl.BlockSpec((C, H), lambda i: (0, 0)),
                pl.BlockSpec((1, C), lambda i: (0, 0)),
            ],
            out_specs=pl.BlockSpec((TB, C), lambda i: (i, 0)),
        ),
        compiler_params=pltpu.CompilerParams(
            dimension_semantics=("parallel",),
            vmem_limit_bytes=64 * 1024 * 1024,
        ),
        cost_estimate=pl.CostEstimate(
            flops=flops, transcendentals=0, bytes_accessed=bytes_accessed),
    )(x, w1b, b1r, w2b, b2r)

    if B_p != B:
        out = out[:B]
    return out


# speedup vs baseline: 1.5361x; 1.0239x over previous
import jax
import jax.numpy as jnp
from jax.experimental import pallas as pl
from jax.experimental.pallas import tpu as pltpu


def _round_up(n, m):
    return ((n + m - 1) // m) * m


def _mlp_fused_kernel(x_ref, w1_ref, b1_ref, w2_ref, b2_ref, o_ref):
    x = x_ref[...].astype(jnp.bfloat16)
    h = jax.lax.dot_general(
        x, w1_ref[...],
        dimension_numbers=(((1,), (1,)), ((), ())),
        preferred_element_type=jnp.float32,
    )
    h = jnp.maximum(h + b1_ref[...], 0.0).astype(jnp.bfloat16)
    out = jax.lax.dot_general(
        h, w2_ref[...],
        dimension_numbers=(((1,), (1,)), ((), ())),
        preferred_element_type=jnp.float32,
    )
    o_ref[...] = (out + b2_ref[...]).astype(o_ref.dtype)


def kernel(x, w1, b1, w2, b2, *, tile_b=1024):
    B, In = x.shape
    H, _ = w1.shape
    C, _ = w2.shape
    dt = x.dtype

    TB = min(tile_b, _round_up(B, 8))
    B_p = _round_up(B, TB)
    if B_p != B:
        x = jnp.pad(x, ((0, B_p - B), (0, 0)))

    w1b = w1.astype(jnp.bfloat16)
    w2b = w2.astype(jnp.bfloat16)
    b1r = b1.reshape(1, H)
    b2r = b2.reshape(1, C)

    grid = (B_p // TB,)
    flops = 2 * B_p * (In * H + H * C)
    bytes_accessed = 4 * (B_p * In + B_p * C + H + C) + 2 * (In * H + H * C)

    out = pl.pallas_call(
        _mlp_fused_kernel,
        out_shape=jax.ShapeDtypeStruct((B_p, C), dt),
        grid_spec=pltpu.PrefetchScalarGridSpec(
            num_scalar_prefetch=0,
            grid=grid,
            in_specs=[
                pl.BlockSpec((TB, In), lambda i: (i, 0)),
                pl.BlockSpec((H, In), lambda i: (0, 0)),
                pl.BlockSpec((1, H), lambda i: (0, 0)),
                pl.BlockSpec((C, H), lambda i: (0, 0)),
                pl.BlockSpec((1, C), lambda i: (0, 0)),
            ],
            out_specs=pl.BlockSpec((TB, C), lambda i: (i, 0)),
        ),
        compiler_params=pltpu.CompilerParams(
            dimension_semantics=("parallel",),
            vmem_limit_bytes=64 * 1024 * 1024,
        ),
        cost_estimate=pl.CostEstimate(
            flops=flops, transcendentals=0, bytes_accessed=bytes_accessed),
    )(x, w1b, b1r, w2b, b2r)

    if B_p != B:
        out = out[:B]
    return out


# speedup vs baseline: 1.6829x; 1.0956x over previous
import jax
import jax.numpy as jnp
from jax.experimental import pallas as pl
from jax.experimental.pallas import tpu as pltpu


def _round_up(n, m):
    return ((n + m - 1) // m) * m


def _mlp_fused_kernel(x_ref, w1_ref, b1_ref, w2_ref, b2_ref, o_ref):
    x = x_ref[...].astype(jnp.bfloat16)
    h = jax.lax.dot_general(
        x, w1_ref[...].astype(jnp.bfloat16),
        dimension_numbers=(((1,), (1,)), ((), ())),
        preferred_element_type=jnp.float32,
    )
    h = jnp.maximum(h + b1_ref[...], 0.0).astype(jnp.bfloat16)
    out = jax.lax.dot_general(
        h, w2_ref[...].astype(jnp.bfloat16),
        dimension_numbers=(((1,), (1,)), ((), ())),
        preferred_element_type=jnp.float32,
    )
    o_ref[...] = (out + b2_ref[...]).astype(o_ref.dtype)


def kernel(x, w1, b1, w2, b2, *, tile_b=1024):
    B, In = x.shape
    H, _ = w1.shape
    C, _ = w2.shape
    dt = x.dtype

    TB = min(tile_b, _round_up(B, 8))
    B_p = _round_up(B, TB)
    if B_p != B:
        x = jnp.pad(x, ((0, B_p - B), (0, 0)))

    b1r = b1.reshape(1, H)
    b2r = b2.reshape(1, C)

    grid = (B_p // TB,)
    flops = 2 * B_p * (In * H + H * C)
    bytes_accessed = 4 * (B_p * In + B_p * C + H + C + In * H + H * C)

    out = pl.pallas_call(
        _mlp_fused_kernel,
        out_shape=jax.ShapeDtypeStruct((B_p, C), dt),
        grid_spec=pltpu.PrefetchScalarGridSpec(
            num_scalar_prefetch=0,
            grid=grid,
            in_specs=[
                pl.BlockSpec((TB, In), lambda i: (i, 0)),
                pl.BlockSpec((H, In), lambda i: (0, 0)),
                pl.BlockSpec((1, H), lambda i: (0, 0)),
                pl.BlockSpec((C, H), lambda i: (0, 0)),
                pl.BlockSpec((1, C), lambda i: (0, 0)),
            ],
            out_specs=pl.BlockSpec((TB, C), lambda i: (i, 0)),
        ),
        compiler_params=pltpu.CompilerParams(
            dimension_semantics=("parallel",),
            vmem_limit_bytes=64 * 1024 * 1024,
        ),
        cost_estimate=pl.CostEstimate(
            flops=flops, transcendentals=0, bytes_accessed=bytes_accessed),
    )(x, w1, b1r, w2, b2r)

    if B_p != B:
        out = out[:B]
    return out
